# baseline (speedup 1.0000x reference)
import functools

import jax
import jax.numpy as jnp
from jax import lax
from jax.experimental import pallas as pl
from jax.experimental.pallas import tpu as pltpu
from jax.experimental.pallas import tpu_sc as plsc

_EDGES = [
    (0, 1), (20, 1), (2, 20), (4, 20), (8, 20), (12, 0), (16, 0), (3, 2),
    (5, 4), (9, 8), (13, 12), (17, 16), (6, 5), (10, 9), (14, 13), (18, 17),
    (7, 6), (11, 10), (15, 14), (19, 18), (21, 7), (22, 7), (23, 11), (24, 11),
]

_NJ = 25
_T = 300
_ROW = _NJ * _T
_L = 16
_C = 8
_NW = 32


_TPL = _L // _C


def _body(bone_hbm, center_hbm, out_hbm, ibuf, obuf, cbuf):
    wid = lax.axis_index("s") * 2 + lax.axis_index("c")
    n_samples = bone_hbm.shape[0] // _ROW
    per_w = n_samples // _NW
    n_chunks = per_w // _C
    base = wid * per_w

    lanes = lax.iota(jnp.int32, _L)
    lanebase = (lanes % _C) * _ROW + lanes // _C
    clanebase = (lanes % _C) * _T + lanes // _C

    def chunk(g, _):
        start = base + g * _C
        pltpu.sync_copy(bone_hbm.at[pl.ds(start * _ROW, _C * _ROW)], ibuf)
        pltpu.sync_copy(center_hbm.at[pl.ds(start * _T, _C * _T)], cbuf)

        @plsc.parallel_loop(0, _T, step=_TPL, unroll=4)
        def compute(t):
            tvec = lanebase + t
            c = plsc.load_gather(cbuf, [clanebase + t])
            vals = {1: c}
            plsc.store_scatter(obuf, [tvec + (1 * _T)], c)
            for v1, v2 in _EDGES:
                idx = tvec + (v1 * _T)
                v = plsc.load_gather(ibuf, [idx]) + vals[v2]
                vals[v1] = v
                plsc.store_scatter(obuf, [idx], v)

        pltpu.sync_copy(obuf, out_hbm.at[pl.ds(start * _ROW, _C * _ROW)])
        return _

    lax.fori_loop(0, n_chunks, chunk, None)


def kernel(bone, center):
    b, ch, nj, t = bone.shape
    n = b * ch
    bone_flat = bone.reshape(n * nj * t)
    center_flat = center.reshape(n * t)

    mesh = plsc.VectorSubcoreMesh(core_axis_name="c", subcore_axis_name="s")
    k = functools.partial(
        pl.kernel,
        out_type=jax.ShapeDtypeStruct((n * nj * t,), jnp.float32),
        mesh=mesh,
        compiler_params=pltpu.CompilerParams(needs_layout_passes=False),
        scratch_types=[
            pltpu.VMEM((_C * _ROW,), jnp.float32),
            pltpu.VMEM((_C * _ROW,), jnp.float32),
            pltpu.VMEM((_C * _T,), jnp.float32),
        ],
    )(_body)
    out = k(bone_flat, center_flat)
    return out.reshape(b, ch, nj, t)


# speedup vs baseline: 1.4990x; 1.4990x over previous
import functools

import jax
import jax.numpy as jnp
from jax import lax
from jax.experimental import pallas as pl
from jax.experimental.pallas import tpu as pltpu
from jax.experimental.pallas import tpu_sc as plsc

_EDGES = [
    (0, 1), (20, 1), (2, 20), (4, 20), (8, 20), (12, 0), (16, 0), (3, 2),
    (5, 4), (9, 8), (13, 12), (17, 16), (6, 5), (10, 9), (14, 13), (18, 17),
    (7, 6), (11, 10), (15, 14), (19, 18), (21, 7), (22, 7), (23, 11), (24, 11),
]

_NJ = 25
_T = 300
_L = 16
_C = 4
_NW = 32
_NB = 6
_COLS = tuple(range(0, _T - _L, _L)) + (_T - _L,)


def _body(bone_hbm, center_hbm, out_hbm, ibuf, obuf, cbuf):
    wid = lax.axis_index("s") * 2 + lax.axis_index("c")
    n_samples = bone_hbm.shape[0]
    per_w = n_samples // _NW
    n_chunks = per_w // _C
    base = wid * per_w

    def chunk(g, _):
        start = base + g * _C
        b0 = start // 3
        pltpu.sync_copy(bone_hbm.at[pl.ds(start, _C)], ibuf)
        pltpu.sync_copy(center_hbm.at[pl.ds(b0, _NB)], cbuf)

        @plsc.parallel_loop(0, _C, unroll=2)
        def compute(s):
            sample = start + s
            bloc = sample // 3 - b0
            c = sample - (sample // 3) * 3
            for col in _COLS:
                cv = cbuf[bloc, c, pl.ds(col, _L)]
                vals = {1: cv}
                obuf[s, 1, pl.ds(col, _L)] = cv
                for v1, v2 in _EDGES:
                    v = ibuf[s, v1, pl.ds(col, _L)] + vals[v2]
                    vals[v1] = v
                    obuf[s, v1, pl.ds(col, _L)] = v

        pltpu.sync_copy(obuf, out_hbm.at[pl.ds(start, _C)])
        return _

    lax.fori_loop(0, n_chunks, chunk, None)


def kernel(bone, center):
    b, ch, nj, t = bone.shape
    n = b * ch
    bone_flat = bone.reshape(n, nj, t)

    mesh = plsc.VectorSubcoreMesh(core_axis_name="c", subcore_axis_name="s")
    k = functools.partial(
        pl.kernel,
        out_type=jax.ShapeDtypeStruct((n, nj, t), jnp.float32),
        mesh=mesh,
        scratch_types=[
            pltpu.VMEM((_C, _NJ, _T), jnp.float32),
            pltpu.VMEM((_C, _NJ, _T), jnp.float32),
            pltpu.VMEM((_NB, ch, _T), jnp.float32),
        ],
    )(_body)
    out = k(bone_flat, center)
    return out.reshape(b, ch, nj, t)
